# initial kernel scaffold (unmeasured)
import jax
import jax.numpy as jnp
from jax import lax
from jax.experimental import pallas as pl
from jax.experimental.pallas import tpu as pltpu


def kernel(
    x,
):
    def body(*refs):
        pass

    out_shape = jax.ShapeDtypeStruct(..., jnp.float32)
    return pl.pallas_call(body, out_shape=out_shape)(...)


# baseline (device time: 196858 ns/iter reference)
import jax
import jax.numpy as jnp
from jax import lax
from jax.experimental import pallas as pl
from jax.experimental.pallas import tpu as pltpu


def kernel(x):
    m, n = x.shape

    def body(x_ref, out_ref, comm_ref, send_sem, recv_sem):
        my_x = lax.axis_index("x")
        my_y = lax.axis_index("y")
        my_z = lax.axis_index("z")
        partner = (my_x, my_y, 1 - my_z)

        barrier_sem = pltpu.get_barrier_semaphore()
        pl.semaphore_signal(
            barrier_sem, inc=1,
            device_id=partner, device_id_type=pl.DeviceIdType.MESH,
        )
        pl.semaphore_wait(barrier_sem, 1)

        rdma = pltpu.make_async_remote_copy(
            src_ref=x_ref,
            dst_ref=comm_ref,
            send_sem=send_sem,
            recv_sem=recv_sem,
            device_id=partner,
            device_id_type=pl.DeviceIdType.MESH,
        )
        rdma.start()
        rdma.wait()

        out_ref[...] = x_ref[...] + comm_ref[...]

    return pl.pallas_call(
        body,
        out_shape=jax.ShapeDtypeStruct((m, n), jnp.float32),
        in_specs=[pl.BlockSpec(memory_space=pltpu.VMEM)],
        out_specs=pl.BlockSpec(memory_space=pltpu.VMEM),
        scratch_shapes=[
            pltpu.VMEM((m, n), jnp.float32),
            pltpu.SemaphoreType.DMA,
            pltpu.SemaphoreType.DMA,
        ],
        compiler_params=pltpu.CompilerParams(collective_id=0),
    )(x)


# device time: 113958 ns/iter; 1.7275x vs baseline; 1.7275x over previous
import jax
import jax.numpy as jnp
from jax import lax
from jax.experimental import pallas as pl
from jax.experimental.pallas import tpu as pltpu

N_CHUNK = 16


def kernel(x):
    m, n = x.shape
    half = m // 2
    rows = half // N_CHUNK

    def body(x_ref, out_ref, comm_ref, z_send, z_recv, y_send, y_recv):
        my_x = lax.axis_index("x")
        my_y = lax.axis_index("y")
        my_z = lax.axis_index("z")
        z_nbr = (my_x, my_y, 1 - my_z)
        y_nbr = (my_x, 1 - my_y, my_z)
        h0 = my_y * half
        o0 = (1 - my_y) * half

        barrier_sem = pltpu.get_barrier_semaphore()
        for nbr in (z_nbr, y_nbr):
            pl.semaphore_signal(
                barrier_sem, inc=1,
                device_id=nbr, device_id_type=pl.DeviceIdType.MESH,
            )
        pl.semaphore_wait(barrier_sem, 2)

        z_rdmas = []
        for k in range(N_CHUNK):
            r = pltpu.make_async_remote_copy(
                src_ref=x_ref.at[pl.ds(h0 + k * rows, rows), :],
                dst_ref=comm_ref.at[pl.ds(k * rows, rows), :],
                send_sem=z_send.at[k],
                recv_sem=z_recv.at[k],
                device_id=z_nbr,
                device_id_type=pl.DeviceIdType.MESH,
            )
            r.start()
            z_rdmas.append(r)

        y_rdmas = []
        for k in range(N_CHUNK):
            z_rdmas[k].wait_recv()
            sl = pl.ds(h0 + k * rows, rows)
            out_ref[sl, :] = x_ref[sl, :] + comm_ref[pl.ds(k * rows, rows), :]
            r = pltpu.make_async_remote_copy(
                src_ref=out_ref.at[sl, :],
                dst_ref=out_ref.at[sl, :],
                send_sem=y_send.at[k],
                recv_sem=y_recv.at[k],
                device_id=y_nbr,
                device_id_type=pl.DeviceIdType.MESH,
            )
            r.start()
            y_rdmas.append(r)

        for k in range(N_CHUNK):
            recv = pltpu.make_async_remote_copy(
                src_ref=out_ref.at[pl.ds(o0 + k * rows, rows), :],
                dst_ref=out_ref.at[pl.ds(o0 + k * rows, rows), :],
                send_sem=y_send.at[k],
                recv_sem=y_recv.at[k],
                device_id=y_nbr,
                device_id_type=pl.DeviceIdType.MESH,
            )
            recv.wait_recv()
        for k in range(N_CHUNK):
            z_rdmas[k].wait_send()
            y_rdmas[k].wait_send()

    return pl.pallas_call(
        body,
        out_shape=jax.ShapeDtypeStruct((m, n), jnp.float32),
        in_specs=[pl.BlockSpec(memory_space=pltpu.VMEM)],
        out_specs=pl.BlockSpec(memory_space=pltpu.VMEM),
        scratch_shapes=[
            pltpu.VMEM((half, n), jnp.float32),
            pltpu.SemaphoreType.DMA((N_CHUNK,)),
            pltpu.SemaphoreType.DMA((N_CHUNK,)),
            pltpu.SemaphoreType.DMA((N_CHUNK,)),
            pltpu.SemaphoreType.DMA((N_CHUNK,)),
        ],
        compiler_params=pltpu.CompilerParams(collective_id=0),
    )(x)


# device time: 107351 ns/iter; 1.8338x vs baseline; 1.0615x over previous
import jax
import jax.numpy as jnp
from jax import lax
from jax.experimental import pallas as pl
from jax.experimental.pallas import tpu as pltpu

N_CHUNK = 32


def kernel(x):
    m, n = x.shape
    half = m // 2
    rows = half // N_CHUNK

    def body(x_hbm, out_hbm, xh_ref, comm_ref,
             z_send, z_recv, y_send, y_recv, lin_sem, lout_sem):
        my_x = lax.axis_index("x")
        my_y = lax.axis_index("y")
        my_z = lax.axis_index("z")
        z_nbr = (my_x, my_y, 1 - my_z)
        y_nbr = (my_x, 1 - my_y, my_z)
        h0 = my_y * half
        o0 = (1 - my_y) * half

        barrier_sem = pltpu.get_barrier_semaphore()
        for nbr in (z_nbr, y_nbr):
            pl.semaphore_signal(
                barrier_sem, inc=1,
                device_id=nbr, device_id_type=pl.DeviceIdType.MESH,
            )
        pl.semaphore_wait(barrier_sem, 2)

        z_rdmas = []
        local_in = []
        for k in range(N_CHUNK):
            src_sl = pl.ds(h0 + k * rows, rows)
            dst_sl = pl.ds(k * rows, rows)
            r = pltpu.make_async_remote_copy(
                src_ref=x_hbm.at[src_sl, :],
                dst_ref=comm_ref.at[dst_sl, :],
                send_sem=z_send.at[k],
                recv_sem=z_recv.at[k],
                device_id=z_nbr,
                device_id_type=pl.DeviceIdType.MESH,
            )
            r.start()
            z_rdmas.append(r)
            c = pltpu.make_async_copy(
                x_hbm.at[src_sl, :], xh_ref.at[dst_sl, :], lin_sem.at[k]
            )
            c.start()
            local_in.append(c)

        y_rdmas = []
        local_out = []
        for k in range(N_CHUNK):
            local_in[k].wait()
            z_rdmas[k].wait_recv()
            sl = pl.ds(k * rows, rows)
            comm_ref[sl, :] = xh_ref[sl, :] + comm_ref[sl, :]
            out_sl = pl.ds(h0 + k * rows, rows)
            r = pltpu.make_async_remote_copy(
                src_ref=comm_ref.at[sl, :],
                dst_ref=out_hbm.at[out_sl, :],
                send_sem=y_send.at[k],
                recv_sem=y_recv.at[k],
                device_id=y_nbr,
                device_id_type=pl.DeviceIdType.MESH,
            )
            r.start()
            y_rdmas.append(r)
            c = pltpu.make_async_copy(
                comm_ref.at[sl, :], out_hbm.at[out_sl, :], lout_sem.at[k]
            )
            c.start()
            local_out.append(c)

        for k in range(N_CHUNK):
            in_sl = pl.ds(o0 + k * rows, rows)
            recv = pltpu.make_async_remote_copy(
                src_ref=out_hbm.at[in_sl, :],
                dst_ref=out_hbm.at[in_sl, :],
                send_sem=y_send.at[k],
                recv_sem=y_recv.at[k],
                device_id=y_nbr,
                device_id_type=pl.DeviceIdType.MESH,
            )
            recv.wait_recv()
        for k in range(N_CHUNK):
            local_out[k].wait()
            z_rdmas[k].wait_send()
            y_rdmas[k].wait_send()

    return pl.pallas_call(
        body,
        out_shape=jax.ShapeDtypeStruct((m, n), jnp.float32),
        in_specs=[pl.BlockSpec(memory_space=pl.ANY)],
        out_specs=pl.BlockSpec(memory_space=pl.ANY),
        scratch_shapes=[
            pltpu.VMEM((half, n), jnp.float32),
            pltpu.VMEM((half, n), jnp.float32),
            pltpu.SemaphoreType.DMA((N_CHUNK,)),
            pltpu.SemaphoreType.DMA((N_CHUNK,)),
            pltpu.SemaphoreType.DMA((N_CHUNK,)),
            pltpu.SemaphoreType.DMA((N_CHUNK,)),
            pltpu.SemaphoreType.DMA((N_CHUNK,)),
            pltpu.SemaphoreType.DMA((N_CHUNK,)),
        ],
        compiler_params=pltpu.CompilerParams(collective_id=0),
    )(x)


# device time: 87753 ns/iter; 2.2433x vs baseline; 1.2233x over previous
import jax
import jax.numpy as jnp
from jax import lax
from jax.experimental import pallas as pl
from jax.experimental.pallas import tpu as pltpu

KQ = 8
KF = KQ // 2


def kernel(x):
    m, n = x.shape
    Q = m // 4
    rows = Q // KQ
    halfq = Q // 2

    def body(x_hbm, out_hbm, xq_ref, comm_ref, gx_ref, gy_ref,
             z_send, z_recv, xa_send, ya_send,
             gx_recv, xo_recv, yo_recv, gy_recv,
             fx_send, fy_send, fxo_recv, fyo_recv,
             lin_sem, lred_sem, lgx_sem, lgy_sem):
        my_x = lax.axis_index("x")
        my_y = lax.axis_index("y")
        my_z = lax.axis_index("z")
        z_nbr = (my_x, my_y, 1 - my_z)
        x_nbr = (1 - my_x, my_y, my_z)
        y_nbr = (my_x, 1 - my_y, my_z)
        q_me = 2 * my_x + my_y
        q_x = 2 * (1 - my_x) + my_y
        q_y = 2 * my_x + (1 - my_y)
        q_d = 2 * (1 - my_x) + (1 - my_y)
        b_me = q_me * Q

        barrier_sem = pltpu.get_barrier_semaphore()
        for nbr in (z_nbr, x_nbr, y_nbr):
            pl.semaphore_signal(
                barrier_sem, inc=1,
                device_id=nbr, device_id_type=pl.DeviceIdType.MESH,
            )
        pl.semaphore_wait(barrier_sem, 3)

        def rdma(src, dst, ssem, rsem, dev):
            return pltpu.make_async_remote_copy(
                src_ref=src, dst_ref=dst, send_sem=ssem, recv_sem=rsem,
                device_id=dev, device_id_type=pl.DeviceIdType.MESH,
            )

        z_rdmas, local_in = [], []
        for k in range(KQ):
            src_sl = pl.ds(b_me + k * rows, rows)
            dst_sl = pl.ds(k * rows, rows)
            r = rdma(x_hbm.at[src_sl, :], comm_ref.at[dst_sl, :],
                     z_send.at[k], z_recv.at[k], z_nbr)
            r.start()
            z_rdmas.append(r)
            c = pltpu.make_async_copy(
                x_hbm.at[src_sl, :], xq_ref.at[dst_sl, :], lin_sem.at[k]
            )
            c.start()
            local_in.append(c)

        xa_rdmas, ya_rdmas, local_red = [], [], []
        for k in range(KQ):
            local_in[k].wait()
            z_rdmas[k].wait_recv()
            sl = pl.ds(k * rows, rows)
            comm_ref[sl, :] = xq_ref[sl, :] + comm_ref[sl, :]
            out_sl = pl.ds(b_me + k * rows, rows)
            c = pltpu.make_async_copy(
                comm_ref.at[sl, :], out_hbm.at[out_sl, :], lred_sem.at[k]
            )
            c.start()
            local_red.append(c)
            if k < KF:
                rx = rdma(comm_ref.at[sl, :], gx_ref.at[sl, :],
                          xa_send.at[k], gx_recv.at[k], x_nbr)
                ry = rdma(comm_ref.at[sl, :], out_hbm.at[out_sl, :],
                          ya_send.at[k], yo_recv.at[k], y_nbr)
            else:
                j = k - KF
                rx = rdma(comm_ref.at[sl, :], out_hbm.at[out_sl, :],
                          xa_send.at[k], xo_recv.at[j], x_nbr)
                ry = rdma(comm_ref.at[sl, :],
                          gy_ref.at[pl.ds(j * rows, rows), :],
                          ya_send.at[k], gy_recv.at[j], y_nbr)
            rx.start()
            ry.start()
            xa_rdmas.append(rx)
            ya_rdmas.append(ry)

        fy_rdmas, local_gx = [], []
        for j in range(KF):
            sl = pl.ds(j * rows, rows)
            out_sl = pl.ds(q_x * Q + j * rows, rows)
            rdma(gx_ref.at[sl, :], gx_ref.at[sl, :],
                 xa_send.at[j], gx_recv.at[j], x_nbr).wait_recv()
            r = rdma(gx_ref.at[sl, :], out_hbm.at[out_sl, :],
                     fy_send.at[j], fyo_recv.at[j], y_nbr)
            r.start()
            fy_rdmas.append(r)
            c = pltpu.make_async_copy(
                gx_ref.at[sl, :], out_hbm.at[out_sl, :], lgx_sem.at[j]
            )
            c.start()
            local_gx.append(c)

        fx_rdmas, local_gy = [], []
        for j in range(KF):
            sl = pl.ds(j * rows, rows)
            out_sl = pl.ds(q_y * Q + halfq + j * rows, rows)
            rdma(gy_ref.at[sl, :], gy_ref.at[sl, :],
                 ya_send.at[KF + j], gy_recv.at[j], y_nbr).wait_recv()
            r = rdma(gy_ref.at[sl, :], out_hbm.at[out_sl, :],
                     fx_send.at[j], fxo_recv.at[j], x_nbr)
            r.start()
            fx_rdmas.append(r)
            c = pltpu.make_async_copy(
                gy_ref.at[sl, :], out_hbm.at[out_sl, :], lgy_sem.at[j]
            )
            c.start()
            local_gy.append(c)

        for j in range(KF):
            sl = pl.ds(q_x * Q + halfq + j * rows, rows)
            rdma(out_hbm.at[sl, :], out_hbm.at[sl, :],
                 xa_send.at[KF + j], xo_recv.at[j], x_nbr).wait_recv()
        for j in range(KF):
            sl = pl.ds(q_y * Q + j * rows, rows)
            rdma(out_hbm.at[sl, :], out_hbm.at[sl, :],
                 ya_send.at[j], yo_recv.at[j], y_nbr).wait_recv()
        for j in range(KF):
            sl = pl.ds(q_d * Q + halfq + j * rows, rows)
            rdma(out_hbm.at[sl, :], out_hbm.at[sl, :],
                 fx_send.at[j], fxo_recv.at[j], x_nbr).wait_recv()
        for j in range(KF):
            sl = pl.ds(q_d * Q + j * rows, rows)
            rdma(out_hbm.at[sl, :], out_hbm.at[sl, :],
                 fy_send.at[j], fyo_recv.at[j], y_nbr).wait_recv()

        for k in range(KQ):
            z_rdmas[k].wait_send()
            xa_rdmas[k].wait_send()
            ya_rdmas[k].wait_send()
            local_red[k].wait()
        for j in range(KF):
            fx_rdmas[j].wait_send()
            fy_rdmas[j].wait_send()
            local_gx[j].wait()
            local_gy[j].wait()

    return pl.pallas_call(
        body,
        out_shape=jax.ShapeDtypeStruct((m, n), jnp.float32),
        in_specs=[pl.BlockSpec(memory_space=pl.ANY)],
        out_specs=pl.BlockSpec(memory_space=pl.ANY),
        scratch_shapes=[
            pltpu.VMEM((Q, n), jnp.float32),
            pltpu.VMEM((Q, n), jnp.float32),
            pltpu.VMEM((halfq, n), jnp.float32),
            pltpu.VMEM((halfq, n), jnp.float32),
            pltpu.SemaphoreType.DMA((KQ,)),
            pltpu.SemaphoreType.DMA((KQ,)),
            pltpu.SemaphoreType.DMA((KQ,)),
            pltpu.SemaphoreType.DMA((KQ,)),
            pltpu.SemaphoreType.DMA((KF,)),
            pltpu.SemaphoreType.DMA((KF,)),
            pltpu.SemaphoreType.DMA((KF,)),
            pltpu.SemaphoreType.DMA((KF,)),
            pltpu.SemaphoreType.DMA((KF,)),
            pltpu.SemaphoreType.DMA((KF,)),
            pltpu.SemaphoreType.DMA((KF,)),
            pltpu.SemaphoreType.DMA((KF,)),
            pltpu.SemaphoreType.DMA((KQ,)),
            pltpu.SemaphoreType.DMA((KQ,)),
            pltpu.SemaphoreType.DMA((KF,)),
            pltpu.SemaphoreType.DMA((KF,)),
        ],
        compiler_params=pltpu.CompilerParams(collective_id=0),
    )(x)
